# baseline (speedup 1.0000x reference)
import functools

import jax
import jax.numpy as jnp
from jax import lax
from jax.experimental import pallas as pl
from jax.experimental.pallas import tpu as pltpu
from jax.experimental.pallas import tpu_sc as plsc

N = 9216
K = 8192
C = 256

TN = 256
TK = 2048
N_TILES = N // TN
K_TILES = K // TK


def _vq_body(x_ref, et_ref, out_ref, bestv_ref, besti_ref):
    k = pl.program_id(0)
    n = pl.program_id(1)
    x = x_ref[...]
    et = et_ref[...]
    m = jnp.dot(x, et, preferred_element_type=jnp.float32)
    a = jnp.sum(x * x, axis=1, keepdims=True)
    c = jnp.sum(et * et, axis=0, keepdims=True)
    d = -(a - 2.0 * m + c)
    tmax = jnp.max(d, axis=1, keepdims=True)
    ids = lax.broadcasted_iota(jnp.int32, (TN, TK), 1)
    loc = jnp.min(jnp.where(d == tmax, ids, TK), axis=1, keepdims=True)
    gidx = k * TK + loc

    sl = pl.ds(n * TN, TN)

    @pl.when(k == 0)
    def _():
        bestv_ref[sl, :] = tmax
        besti_ref[sl, :] = gidx

    @pl.when(k > 0)
    def _():
        bv = bestv_ref[sl, :]
        bi = besti_ref[sl, :]
        upd = tmax > bv
        bestv_ref[sl, :] = jnp.where(upd, tmax, bv)
        besti_ref[sl, :] = jnp.where(upd, gidx, bi)

    out_ref[...] = besti_ref[sl, :]


def _vq_argmax(flat, embed_t):
    return pl.pallas_call(
        _vq_body,
        grid=(K_TILES, N_TILES),
        in_specs=[
            pl.BlockSpec((TN, C), lambda k, n: (n, 0)),
            pl.BlockSpec((C, TK), lambda k, n: (0, k)),
        ],
        out_specs=pl.BlockSpec((TN, 1), lambda k, n: (n, 0)),
        out_shape=jax.ShapeDtypeStruct((N, 1), jnp.int32),
        scratch_shapes=[
            pltpu.VMEM((N, 1), jnp.float32),
            pltpu.VMEM((N, 1), jnp.int32),
        ],
    )(flat, embed_t)


_info = plsc.get_sparse_core_info()
_NC, _NS = _info.num_cores, _info.num_subcores
_NW = _NC * _NS
_BPW = N // _NW
_GCH = 96
_NCH = _BPW // _GCH


@functools.partial(
    pl.kernel,
    mesh=plsc.VectorSubcoreMesh(core_axis_name="c", subcore_axis_name="s"),
    out_type=jax.ShapeDtypeStruct((N, C), jnp.float32),
    scratch_types=[
        pltpu.VMEM((_BPW,), jnp.int32),
        pltpu.VMEM((_BPW, C), jnp.float32),
        pltpu.SemaphoreType.DMA,
    ],
)
def _sc_gather(table_hbm, idx_hbm, out_hbm, idx_v, rows_v, sem):
    wid = lax.axis_index("s") * _NC + lax.axis_index("c")
    base = wid * _BPW
    pltpu.sync_copy(idx_hbm.at[pl.ds(base, _BPW)], idx_v)
    copies = []
    for j in range(_NCH):
        copies.append(pltpu.async_copy(
            table_hbm.at[idx_v.at[pl.ds(j * _GCH, _GCH)]],
            rows_v.at[pl.ds(j * _GCH, _GCH)],
            sem,
        ))
    for cp in copies:
        cp.wait()
    pltpu.sync_copy(rows_v, out_hbm.at[pl.ds(base, _BPW)])


def kernel(x, embed):
    Bb, Tt, Cc = x.shape
    flat = x.reshape(Bb * Tt, Cc)
    idx = _vq_argmax(flat, embed.T).reshape(-1)
    quant = _sc_gather(embed, idx)
    return quant.reshape(Bb, Tt, Cc)


# speedup vs baseline: 1.0272x; 1.0272x over previous
import functools

import jax
import jax.numpy as jnp
from jax import lax
from jax.experimental import pallas as pl
from jax.experimental.pallas import tpu as pltpu
from jax.experimental.pallas import tpu_sc as plsc

N = 9216
K = 8192
C = 256

TN = 256
TK = 2048
N_TILES = N // TN
K_TILES = K // TK


def _vq_body(x_ref, et_ref, out_ref, bestv_ref, besti_ref, c_ref):
    k = pl.program_id(0)
    n = pl.program_id(1)
    x = x_ref[...]
    et = et_ref[...]
    m2 = jnp.dot(x + x, et, preferred_element_type=jnp.float32)
    a = jnp.sum(x * x, axis=1, keepdims=True)

    @pl.when(n == 0)
    def _():
        c_ref[...] = jnp.sum(et * et, axis=0, keepdims=True)

    c = c_ref[...]
    t = a - m2 + c
    tmin = jnp.min(t, axis=1, keepdims=True)
    ids = lax.broadcasted_iota(jnp.int32, (TN, TK), 1)
    loc = jnp.min(jnp.where(t == tmin, ids, TK), axis=1, keepdims=True)
    gidx = k * TK + loc

    sl = pl.ds(n * TN, TN)

    @pl.when(k == 0)
    def _():
        bestv_ref[sl, :] = tmin
        besti_ref[sl, :] = gidx

    @pl.when(k > 0)
    def _():
        bv = bestv_ref[sl, :]
        bi = besti_ref[sl, :]
        upd = tmin < bv
        bestv_ref[sl, :] = jnp.where(upd, tmin, bv)
        besti_ref[sl, :] = jnp.where(upd, gidx, bi)

    out_ref[...] = besti_ref[sl, :]


def _vq_argmax(flat, embed_t):
    return pl.pallas_call(
        _vq_body,
        grid=(K_TILES, N_TILES),
        in_specs=[
            pl.BlockSpec((TN, C), lambda k, n: (n, 0)),
            pl.BlockSpec((C, TK), lambda k, n: (0, k)),
        ],
        out_specs=pl.BlockSpec((TN, 1), lambda k, n: (n, 0)),
        out_shape=jax.ShapeDtypeStruct((N, 1), jnp.int32),
        scratch_shapes=[
            pltpu.VMEM((N, 1), jnp.float32),
            pltpu.VMEM((N, 1), jnp.int32),
            pltpu.VMEM((1, TK), jnp.float32),
        ],
    )(flat, embed_t)


_info = plsc.get_sparse_core_info()
_NC, _NS = _info.num_cores, _info.num_subcores
_NW = _NC * _NS
_BPW = N // _NW
_GCH = 96
_NCH = _BPW // _GCH


@functools.partial(
    pl.kernel,
    mesh=plsc.VectorSubcoreMesh(core_axis_name="c", subcore_axis_name="s"),
    out_type=jax.ShapeDtypeStruct((N, C), jnp.float32),
    scratch_types=[
        pltpu.VMEM((_BPW,), jnp.int32),
        pltpu.VMEM((_BPW, C), jnp.float32),
        pltpu.SemaphoreType.DMA,
    ],
)
def _sc_gather(table_hbm, idx_hbm, out_hbm, idx_v, rows_v, sem):
    wid = lax.axis_index("s") * _NC + lax.axis_index("c")
    base = wid * _BPW
    pltpu.sync_copy(idx_hbm.at[pl.ds(base, _BPW)], idx_v)
    copies = []
    for j in range(_NCH):
        copies.append(pltpu.async_copy(
            table_hbm.at[idx_v.at[pl.ds(j * _GCH, _GCH)]],
            rows_v.at[pl.ds(j * _GCH, _GCH)],
            sem,
        ))
    for cp in copies:
        cp.wait()
    pltpu.sync_copy(rows_v, out_hbm.at[pl.ds(base, _BPW)])


def kernel(x, embed):
    Bb, Tt, Cc = x.shape
    flat = x.reshape(Bb * Tt, Cc)
    idx = _vq_argmax(flat, embed.T).reshape(-1)
    quant = _sc_gather(embed, idx)
    return quant.reshape(Bb, Tt, Cc)


# speedup vs baseline: 1.5705x; 1.5290x over previous
import functools

import jax
import jax.numpy as jnp
from jax import lax
from jax.experimental import pallas as pl
from jax.experimental.pallas import tpu as pltpu
from jax.experimental.pallas import tpu_sc as plsc

N = 9216
K = 8192
C = 256

TN = 128
TG = 2048
CH = 128
N_TILES = N // TN
N_SUB = K // TG
N_CH = TG // CH


def _vq_body(x_ref, et_ref, out_ref, c_ref):
    n = pl.program_id(0)
    x = x_ref[...]

    @pl.when(n == 0)
    def _():
        c_ref[...] = jnp.sum(et_ref[...] * et_ref[...], axis=0, keepdims=True)

    x2 = x + x
    a = jnp.sum(x * x, axis=1, keepdims=True)

    bv = None
    bi = None
    for g in range(N_SUB):
        et_g = et_ref[:, pl.ds(g * TG, TG)]
        m2 = jnp.dot(x2, et_g, preferred_element_type=jnp.float32)
        for j in range(N_CH):
            mj = lax.slice(m2, (0, j * CH), (TN, (j + 1) * CH))
            cj = c_ref[0, pl.ds(g * TG + j * CH, CH)]
            t = a - mj + cj
            base = g * TG + j * CH
            if bv is None:
                bv = t
                bi = jnp.zeros((TN, CH), jnp.int32)
            else:
                upd = t < bv
                bv = jnp.where(upd, t, bv)
                bi = jnp.where(upd, base, bi)

    gmin = jnp.min(bv, axis=1, keepdims=True)
    lane = lax.broadcasted_iota(jnp.int32, (TN, CH), 1)
    cand = jnp.where(bv == gmin, bi + lane, K)
    out_ref[...] = jnp.min(cand, axis=1, keepdims=True)


def _vq_argmax(flat, embed_t):
    return pl.pallas_call(
        _vq_body,
        grid=(N_TILES,),
        in_specs=[
            pl.BlockSpec((TN, C), lambda n: (n, 0)),
            pl.BlockSpec((C, K), lambda n: (0, 0)),
        ],
        out_specs=pl.BlockSpec((TN, 1), lambda n: (n, 0)),
        out_shape=jax.ShapeDtypeStruct((N, 1), jnp.int32),
        scratch_shapes=[
            pltpu.VMEM((1, K), jnp.float32),
        ],
    )(flat, embed_t)


_info = plsc.get_sparse_core_info()
_NC, _NS = _info.num_cores, _info.num_subcores
_NW = _NC * _NS
_BPW = N // _NW
_GCH = 96
_NCH = _BPW // _GCH


@functools.partial(
    pl.kernel,
    mesh=plsc.VectorSubcoreMesh(core_axis_name="c", subcore_axis_name="s"),
    out_type=jax.ShapeDtypeStruct((N, C), jnp.float32),
    scratch_types=[
        pltpu.VMEM((_BPW,), jnp.int32),
        pltpu.VMEM((_BPW, C), jnp.float32),
        pltpu.SemaphoreType.DMA,
    ],
)
def _sc_gather(table_hbm, idx_hbm, out_hbm, idx_v, rows_v, sem):
    wid = lax.axis_index("s") * _NC + lax.axis_index("c")
    base = wid * _BPW
    pltpu.sync_copy(idx_hbm.at[pl.ds(base, _BPW)], idx_v)
    copies = []
    for j in range(_NCH):
        copies.append(pltpu.async_copy(
            table_hbm.at[idx_v.at[pl.ds(j * _GCH, _GCH)]],
            rows_v.at[pl.ds(j * _GCH, _GCH)],
            sem,
        ))
    for cp in copies:
        cp.wait()
    pltpu.sync_copy(rows_v, out_hbm.at[pl.ds(base, _BPW)])


def kernel(x, embed):
    Bb, Tt, Cc = x.shape
    flat = x.reshape(Bb * Tt, Cc)
    idx = _vq_argmax(flat, embed.T).reshape(-1)
    quant = _sc_gather(embed, idx)
    return quant.reshape(Bb, Tt, Cc)
